# baseline (speedup 1.0000x reference)
import functools

import jax
import jax.numpy as jnp
import numpy as np
from jax import lax
from jax.experimental import pallas as pl
from jax.experimental.pallas import tpu as pltpu
from jax.experimental.pallas import tpu_sc as plsc

_N = 10000
_E = 320000
_D = 128
_DW = _D // 2

_NC = 2
_NS = 16
_NW = _NC * _NS
_CH = 128
_EPW = 10240
_NCHUNK = _EPW // _CH
_NPAD = _EPW - _E // _NW
_NDUMMY = 16
_NA = _N + _NDUMMY
_RSTEP = 624
_RCNT = 640

_PERM = np.concatenate([np.arange(0, _D, 2), np.arange(1, _D, 2)])
_IPERM = np.argsort(_PERM)


def _sc_segment_sum(xnp, src_w, dstp_w, zrows):
    mesh = plsc.VectorSubcoreMesh(core_axis_name="c", subcore_axis_name="s")

    @functools.partial(
        pl.kernel,
        out_type=jax.ShapeDtypeStruct((_NC, _N, _D), jnp.float32),
        mesh=mesh,
        compiler_params=pltpu.CompilerParams(use_tc_tiling_on_sc=False),
        scratch_types=[
            pltpu.VMEM_SHARED((_NA, _D), jnp.float32),
            pltpu.VMEM((_NCHUNK, _CH), jnp.int32),
            pltpu.VMEM((_NCHUNK * _CH // 2,), jnp.int32),
            pltpu.VMEM((8, _CH // 2), jnp.int32),
            pltpu.VMEM((_CH, _DW), jnp.int32),
            pltpu.VMEM((_CH, _DW), jnp.int32),
            pltpu.VMEM((_CH // 2, _D), jnp.float32),
            pltpu.VMEM((_CH // 2, _D), jnp.float32),
            pltpu.SemaphoreType.DMA,
            pltpu.SemaphoreType.DMA,
            pltpu.SemaphoreType.DMA,
        ],
    )
    def seg_sum(xn_hbm, src_hbm, dstp_hbm, z_hbm, out_hbm,
                acc, src_v, dstp_v, scat_v, gbuf_a, gbuf_b, stag0, stag1,
                sem_a, sem_b, sem_s):
        c = lax.axis_index("c")
        s = lax.axis_index("s")
        wid = s * _NC + c
        stag = (stag0, stag1)
        pltpu.sync_copy(z_hbm, acc.at[pl.ds(s * _RSTEP, _RCNT)])
        pltpu.sync_copy(src_hbm.at[wid], src_v)
        pltpu.sync_copy(dstp_hbm.at[wid], dstp_v)
        plsc.subcore_barrier()

        def unpack_dst(j, h):
            base = j * (_CH // 2) + h * (_CH // 4)
            for t in range(2):
                v = dstp_v[pl.ds(base + t * 16, 16)]
                scat_v[h, pl.ds(t * 16, 16)] = v & 0xFFFF
                scat_v[h, pl.ds(32 + t * 16, 16)] = lax.shift_right_logical(v, 16)

        def convert_half(g_ref, h, s_ref):
            def crow(r, carry):
                for t in range(_DW // 16):
                    w = g_ref[64 * h + r, pl.ds(16 * t, 16)]
                    ev = lax.bitcast_convert_type(lax.shift_left(w, 16), jnp.float32)
                    od = lax.bitcast_convert_type(w & jnp.int32(-65536), jnp.float32)
                    s_ref[r, pl.ds(16 * t, 16)] = ev
                    s_ref[r, pl.ds(_DW + 16 * t, 16)] = od
                return carry

            lax.fori_loop(0, _CH // 2, crow, 0)

        def gather(j, g_ref, sem):
            pltpu.async_copy(xn_hbm.at[src_v.at[j]], g_ref, sem)

        def wait_gather(j, g_ref, sem):
            pltpu.make_async_copy(xn_hbm.at[src_v.at[j]], g_ref, sem).wait()

        def scatter(h):
            pltpu.async_copy(stag[h], acc.at[scat_v.at[h]], sem_s, add=True)

        def wait_scatter(h):
            pltpu.make_async_copy(stag[h], acc.at[scat_v.at[h]], sem_s).wait()

        def chunk_body(j, g_ref, sem, nxt, nxt_ref, nxt_sem, first):
            wait_gather(j, g_ref, sem)
            if nxt is not None:
                gather(nxt, nxt_ref, nxt_sem)
            for h in (0, 1):
                if not first:
                    wait_scatter(h)
                unpack_dst(j, h)
                convert_half(g_ref, h, stag[h])
                scatter(h)

        gather(0, gbuf_a, sem_a)
        chunk_body(0, gbuf_a, sem_a, 1, gbuf_b, sem_b, True)

        def body(i, carry):
            j = 2 * i + 1
            chunk_body(j, gbuf_b, sem_b, j + 1, gbuf_a, sem_a, False)
            chunk_body(j + 1, gbuf_a, sem_a, j + 2, gbuf_b, sem_b, False)
            return carry

        lax.fori_loop(0, (_NCHUNK - 2) // 2, body, 0)
        chunk_body(_NCHUNK - 1, gbuf_b, sem_b, None, None, None, False)
        wait_scatter(0)
        wait_scatter(1)
        plsc.subcore_barrier()
        pltpu.sync_copy(acc.at[pl.ds(s * _RSTEP, _RCNT)],
                        out_hbm.at[c, pl.ds(s * _RSTEP, _RCNT)])

    return seg_sum(xnp, src_w, dstp_w, zrows)


_R = 2000


def _dot(a, b):
    return lax.dot_general(a, b, (((1,), (0,)), ((), ())),
                           precision=lax.Precision.HIGHEST,
                           preferred_element_type=jnp.float32)


def _tc_matmul(x, w):
    def body(x_ref, w_ref, o_ref):
        o_ref[...] = _dot(x_ref[...], w_ref[...]).astype(jnp.bfloat16)

    return pl.pallas_call(
        body,
        grid=(_N // _R,),
        in_specs=[pl.BlockSpec((_R, _D), lambda i: (i, 0)),
                  pl.BlockSpec((_D, _D), lambda i: (0, 0))],
        out_specs=pl.BlockSpec((_R, _D), lambda i: (i, 0)),
        out_shape=jax.ShapeDtypeStruct((_N, _D), jnp.bfloat16),
    )(x, w)


def _tc_combine(x, agg2, w_self, b, gamma, beta, w_next):
    has_next = w_next is not None

    def body(x_ref, agg_ref, ws_ref, b_ref, g_ref, be_ref, *rest):
        if has_next:
            wn_ref, h_ref, y_ref = rest
        else:
            (h_ref,) = rest
        t = (_dot(x_ref[...], ws_ref[...])
             + agg_ref[0] + agg_ref[1] + b_ref[...])
        mu = jnp.mean(t, axis=-1, keepdims=True)
        d = t - mu
        var = jnp.mean(d * d, axis=-1, keepdims=True)
        h = d * lax.rsqrt(var + 1e-5) * g_ref[...] + be_ref[...]
        h = jnp.maximum(h, 0.0)
        h_ref[...] = h
        if has_next:
            y_ref[...] = _dot(h, wn_ref[...]).astype(jnp.bfloat16)

    row_spec = pl.BlockSpec((_R, _D), lambda i: (i, 0))
    full_spec = pl.BlockSpec((_D, _D), lambda i: (0, 0))
    vec_spec = pl.BlockSpec((1, _D), lambda i: (0, 0))
    in_specs = [row_spec,
                pl.BlockSpec((2, _R, _D), lambda i: (0, i, 0)),
                full_spec, vec_spec, vec_spec, vec_spec]
    args = [x, agg2, w_self, b, gamma, beta]
    out_shape = jax.ShapeDtypeStruct((_N, _D), jnp.float32)
    if has_next:
        in_specs.append(full_spec)
        args.append(w_next)
        return pl.pallas_call(
            body,
            grid=(_N // _R,),
            in_specs=in_specs,
            out_specs=(row_spec, row_spec),
            out_shape=(out_shape,
                       jax.ShapeDtypeStruct((_N, _D), jnp.bfloat16)),
        )(*args)
    return pl.pallas_call(
        body,
        grid=(_N // _R,),
        in_specs=in_specs,
        out_specs=row_spec,
        out_shape=out_shape,
    )(*args)


def _pack_rows(xn_bf):
    return lax.bitcast_convert_type(xn_bf.reshape(_N, _DW, 2), jnp.int32)


def kernel(features, edges,
           W_self_0, W_neigh_0, b_0, gamma_0, beta_0,
           W_self_1, W_neigh_1, b_1, gamma_1, beta_1,
           W_self_2, W_neigh_2, b_2, gamma_2, beta_2):
    i_pad = lax.broadcasted_iota(jnp.int32, (_NW, _NPAD), 1)
    w_pad = lax.broadcasted_iota(jnp.int32, (_NW, _NPAD), 0)
    pad_src = (w_pad * 997 + i_pad * 13) % _N
    pad_dst = _N + (i_pad % _NDUMMY)
    src_w = jnp.concatenate(
        [edges[0].reshape(_NW, _E // _NW), pad_src], axis=1
    ).reshape(_NW, _NCHUNK, _CH)
    dst4 = jnp.concatenate(
        [edges[1].reshape(_NW, _E // _NW), pad_dst], axis=1
    ).reshape(_NW, _NCHUNK, 2, 2, _CH // 4)
    dstp_w = (dst4[:, :, :, 0] | (dst4[:, :, :, 1] << 16)
              ).reshape(_NW, _NCHUNK * _CH // 2)
    zrows = jnp.zeros((_RCNT, _D), jnp.float32)

    p = _PERM
    ws0, b0 = W_self_0[:, p], b_0[p].reshape(1, _D)
    g0, be0 = gamma_0[p].reshape(1, _D), beta_0[p].reshape(1, _D)
    ws1, b1 = W_self_1[p][:, p], b_1[p].reshape(1, _D)
    g1, be1 = gamma_1[p].reshape(1, _D), beta_1[p].reshape(1, _D)
    ws2, b2 = W_self_2[p][:, p], b_2[p].reshape(1, _D)
    g2, be2 = gamma_2[p].reshape(1, _D), beta_2[p].reshape(1, _D)
    wn1, wn2 = W_neigh_1[p], W_neigh_2[p]

    xn0 = _tc_matmul(features, W_neigh_0)
    agg0 = _sc_segment_sum(_pack_rows(xn0), src_w, dstp_w, zrows)
    h1, xn1 = _tc_combine(features, agg0, ws0, b0, g0, be0, wn1)
    agg1 = _sc_segment_sum(_pack_rows(xn1), src_w, dstp_w, zrows)
    h2, xn2 = _tc_combine(h1, agg1, ws1, b1, g1, be1, wn2)
    agg2 = _sc_segment_sum(_pack_rows(xn2), src_w, dstp_w, zrows)
    h3p = _tc_combine(h2, agg2, ws2, b2, g2, be2, None)
    return h3p[:, _IPERM]


# speedup vs baseline: 1.7537x; 1.7537x over previous
import functools

import jax
import jax.numpy as jnp
from jax import lax
from jax.experimental import pallas as pl
from jax.experimental.pallas import tpu as pltpu
from jax.experimental.pallas import tpu_sc as plsc

_N = 10000
_E = 320000
_D = 128

_NC = 2
_NS = 16
_NW = _NC * _NS
_CH = 128
_EPW = 10240
_NCHUNK = _EPW // _CH
_NPAD = _EPW - _E // _NW
_NDUMMY = 16
_NA = _N + _NDUMMY
_RSTEP = 624
_RCNT = 640


def _sc_segment_sum(xn, src_w, dstp_w, zrows):
    mesh = plsc.VectorSubcoreMesh(core_axis_name="c", subcore_axis_name="s")

    @functools.partial(
        pl.kernel,
        out_type=(jax.ShapeDtypeStruct((_N, _D), jnp.float32),
                  jax.ShapeDtypeStruct((_N, _D), jnp.float32)),
        mesh=mesh,
        scratch_types=[
            pltpu.VMEM_SHARED((_NA, _D), jnp.float32),
            pltpu.VMEM((_NCHUNK, _CH), jnp.int32),
            pltpu.VMEM((_NCHUNK * _CH // 2,), jnp.int32),
            pltpu.VMEM((8, _CH), jnp.int32),
            pltpu.VMEM((_CH, _D), jnp.float32),
            pltpu.VMEM((_CH, _D), jnp.float32),
            pltpu.SemaphoreType.DMA,
            pltpu.SemaphoreType.DMA,
            pltpu.SemaphoreType.DMA,
        ],
    )
    def seg_sum(xn_hbm, src_hbm, dstp_hbm, z_hbm, out0_hbm, out1_hbm,
                acc, src_v, dstp_v, scat_v, rows_a, rows_b, sem_a, sem_b, sem_s):
        c = lax.axis_index("c")
        s = lax.axis_index("s")
        wid = s * _NC + c
        pltpu.sync_copy(z_hbm, acc.at[pl.ds(s * _RSTEP, _RCNT)])
        pltpu.sync_copy(src_hbm.at[wid], src_v)
        pltpu.sync_copy(dstp_hbm.at[wid], dstp_v)
        plsc.subcore_barrier()

        def unpack_dst(j, p):
            base = j * (_CH // 2)
            for t in range(_CH // 32):
                v = dstp_v[pl.ds(base + t * 16, 16)]
                scat_v[p, pl.ds(t * 16, 16)] = v & 0xFFFF
                scat_v[p, pl.ds(_CH // 2 + t * 16, 16)] = lax.shift_right_logical(v, 16)

        def gather(j, rows, sem):
            pltpu.async_copy(xn_hbm.at[src_v.at[j]], rows, sem)

        def wait_gather(j, rows, sem):
            pltpu.make_async_copy(xn_hbm.at[src_v.at[j]], rows, sem).wait()

        def scatter(rows, p):
            pltpu.async_copy(rows, acc.at[scat_v.at[p]], sem_s, add=True)

        def wait_scatter(rows, p):
            pltpu.make_async_copy(rows, acc.at[scat_v.at[p]], sem_s).wait()

        gather(0, rows_a, sem_a)
        wait_gather(0, rows_a, sem_a)
        unpack_dst(0, 0)
        gather(1, rows_b, sem_b)
        scatter(rows_a, 0)

        def body(i, carry):
            j = 2 * i + 1
            wait_gather(j, rows_b, sem_b)
            unpack_dst(j, 1)
            wait_scatter(rows_a, 0)
            gather(j + 1, rows_a, sem_a)
            scatter(rows_b, 1)
            wait_gather(j + 1, rows_a, sem_a)
            unpack_dst(j + 1, 0)
            wait_scatter(rows_b, 1)
            gather(j + 2, rows_b, sem_b)
            scatter(rows_a, 0)
            return carry

        lax.fori_loop(0, (_NCHUNK - 2) // 2, body, 0)
        j_last = _NCHUNK - 1
        wait_gather(j_last, rows_b, sem_b)
        unpack_dst(j_last, 1)
        wait_scatter(rows_a, 0)
        scatter(rows_b, 1)
        wait_scatter(rows_b, 1)
        plsc.subcore_barrier()

        @pl.when(c == 0)
        def _():
            pltpu.sync_copy(acc.at[pl.ds(s * _RSTEP, _RCNT)],
                            out0_hbm.at[pl.ds(s * _RSTEP, _RCNT)])

        @pl.when(c == 1)
        def _():
            pltpu.sync_copy(acc.at[pl.ds(s * _RSTEP, _RCNT)],
                            out1_hbm.at[pl.ds(s * _RSTEP, _RCNT)])

    return seg_sum(xn, src_w, dstp_w, zrows)


_R = 2000


def _dot(a, b):
    return lax.dot_general(a, b, (((1,), (0,)), ((), ())),
                           precision=lax.Precision.HIGHEST,
                           preferred_element_type=jnp.float32)


def _tc_matmul(x, w):
    def body(x_ref, w_ref, o_ref):
        o_ref[...] = _dot(x_ref[...], w_ref[...])

    return pl.pallas_call(
        body,
        grid=(_N // _R,),
        in_specs=[pl.BlockSpec((_R, _D), lambda i: (i, 0)),
                  pl.BlockSpec((_D, _D), lambda i: (0, 0))],
        out_specs=pl.BlockSpec((_R, _D), lambda i: (i, 0)),
        out_shape=jax.ShapeDtypeStruct((_N, _D), jnp.float32),
    )(x, w)


def _tc_combine(x, agg2, w_self, b, gamma, beta, w_next):
    agg_a, agg_b = agg2
    has_next = w_next is not None

    def body(x_ref, agga_ref, aggb_ref, ws_ref, b_ref, g_ref, be_ref, *rest):
        if has_next:
            wn_ref, h_ref, y_ref = rest
        else:
            (h_ref,) = rest
        t = (_dot(x_ref[...], ws_ref[...])
             + agga_ref[...] + aggb_ref[...] + b_ref[...])
        mu = jnp.mean(t, axis=-1, keepdims=True)
        d = t - mu
        var = jnp.mean(d * d, axis=-1, keepdims=True)
        h = d * lax.rsqrt(var + 1e-5) * g_ref[...] + be_ref[...]
        h = jnp.maximum(h, 0.0)
        h_ref[...] = h
        if has_next:
            y_ref[...] = _dot(h, wn_ref[...])

    row_spec = pl.BlockSpec((_R, _D), lambda i: (i, 0))
    full_spec = pl.BlockSpec((_D, _D), lambda i: (0, 0))
    vec_spec = pl.BlockSpec((1, _D), lambda i: (0, 0))
    in_specs = [row_spec, row_spec, row_spec,
                full_spec, vec_spec, vec_spec, vec_spec]
    args = [x, agg_a, agg_b, w_self, b, gamma, beta]
    out_shape = jax.ShapeDtypeStruct((_N, _D), jnp.float32)
    if has_next:
        in_specs.append(full_spec)
        args.append(w_next)
        return pl.pallas_call(
            body,
            grid=(_N // _R,),
            in_specs=in_specs,
            out_specs=(row_spec, row_spec),
            out_shape=(out_shape, out_shape),
        )(*args)
    return pl.pallas_call(
        body,
        grid=(_N // _R,),
        in_specs=in_specs,
        out_specs=row_spec,
        out_shape=out_shape,
    )(*args)


def kernel(features, edges,
           W_self_0, W_neigh_0, b_0, gamma_0, beta_0,
           W_self_1, W_neigh_1, b_1, gamma_1, beta_1,
           W_self_2, W_neigh_2, b_2, gamma_2, beta_2):
    i_pad = lax.broadcasted_iota(jnp.int32, (_NW, _NPAD), 1)
    w_pad = lax.broadcasted_iota(jnp.int32, (_NW, _NPAD), 0)
    pad_src = (w_pad * 997 + i_pad * 13) % _N
    pad_dst = _N + (i_pad % _NDUMMY)
    src_w = jnp.concatenate(
        [edges[0].reshape(_NW, _E // _NW), pad_src], axis=1
    ).reshape(_NW, _NCHUNK, _CH)
    dst3 = jnp.concatenate(
        [edges[1].reshape(_NW, _E // _NW), pad_dst], axis=1
    ).reshape(_NW, _NCHUNK, _CH)
    dstp_w = (dst3[:, :, : _CH // 2] | (dst3[:, :, _CH // 2:] << 16)
              ).reshape(_NW, _NCHUNK * _CH // 2)
    zrows = jnp.zeros((_RCNT, _D), jnp.float32)
    b0, g0, be0 = b_0.reshape(1, _D), gamma_0.reshape(1, _D), beta_0.reshape(1, _D)
    b1, g1, be1 = b_1.reshape(1, _D), gamma_1.reshape(1, _D), beta_1.reshape(1, _D)
    b2, g2, be2 = b_2.reshape(1, _D), gamma_2.reshape(1, _D), beta_2.reshape(1, _D)

    xn0 = _tc_matmul(features, W_neigh_0)
    agg0 = _sc_segment_sum(xn0, src_w, dstp_w, zrows)
    h1, xn1 = _tc_combine(features, agg0, W_self_0, b0, g0, be0, W_neigh_1)
    agg1 = _sc_segment_sum(xn1, src_w, dstp_w, zrows)
    h2, xn2 = _tc_combine(h1, agg1, W_self_1, b1, g1, be1, W_neigh_2)
    agg2 = _sc_segment_sum(xn2, src_w, dstp_w, zrows)
    return _tc_combine(h2, agg2, W_self_2, b2, g2, be2, None)


# speedup vs baseline: 1.7706x; 1.0096x over previous
import functools

import jax
import jax.numpy as jnp
from jax import lax
from jax.experimental import pallas as pl
from jax.experimental.pallas import tpu as pltpu
from jax.experimental.pallas import tpu_sc as plsc

_N = 10000
_E = 320000
_D = 128

_NC = 2
_NS = 16
_NW = _NC * _NS
_CH = 128
_EPW = 10240
_NCHUNK = _EPW // _CH
_NPAD = _EPW - _E // _NW
_NDUMMY = 16
_NA = _N + _NDUMMY
_RSTEP = 624
_RCNT = 640


def _sc_segment_sum(xn, src_w, dstp_w, zrows):
    mesh = plsc.VectorSubcoreMesh(core_axis_name="c", subcore_axis_name="s")

    @functools.partial(
        pl.kernel,
        out_type=(jax.ShapeDtypeStruct((_N, _D), jnp.float32),
                  jax.ShapeDtypeStruct((_N, _D), jnp.float32)),
        mesh=mesh,
        scratch_types=[
            pltpu.VMEM_SHARED((_NA, _D), jnp.float32),
            pltpu.VMEM((_NCHUNK, _CH), jnp.int32),
            pltpu.VMEM((_NCHUNK * _CH // 2,), jnp.int32),
            pltpu.VMEM((8, _CH), jnp.int32),
            pltpu.VMEM((_CH, _D), jnp.float32),
            pltpu.VMEM((_CH, _D), jnp.float32),
            pltpu.SemaphoreType.DMA,
            pltpu.SemaphoreType.DMA,
            pltpu.SemaphoreType.DMA,
        ],
    )
    def seg_sum(xn_hbm, src_hbm, dstp_hbm, z_hbm, out0_hbm, out1_hbm,
                acc, src_v, dstp_v, scat_v, rows_a, rows_b, sem_a, sem_b, sem_s):
        c = lax.axis_index("c")
        s = lax.axis_index("s")
        wid = s * _NC + c
        pltpu.sync_copy(z_hbm, acc.at[pl.ds(s * _RSTEP, _RCNT)])
        pltpu.sync_copy(src_hbm.at[wid], src_v)
        pltpu.sync_copy(dstp_hbm.at[wid], dstp_v)
        plsc.subcore_barrier()

        def unpack_dst(j, p):
            base = j * (_CH // 2)
            for t in range(_CH // 32):
                v = dstp_v[pl.ds(base + t * 16, 16)]
                scat_v[p, pl.ds(t * 16, 16)] = v & 0xFFFF
                scat_v[p, pl.ds(_CH // 2 + t * 16, 16)] = lax.shift_right_logical(v, 16)

        def gather(j, rows, sem):
            pltpu.async_copy(xn_hbm.at[src_v.at[j]], rows, sem)

        def wait_gather(j, rows, sem):
            pltpu.make_async_copy(xn_hbm.at[src_v.at[j]], rows, sem).wait()

        def scatter(rows, p):
            pltpu.async_copy(rows, acc.at[scat_v.at[p]], sem_s, add=True)

        def wait_scatter(rows, p):
            pltpu.make_async_copy(rows, acc.at[scat_v.at[p]], sem_s).wait()

        def sync_scatter(rows, p):
            pltpu.sync_copy(rows, acc.at[scat_v.at[p]], add=True)

        gather(0, rows_a, sem_a)

        def two_chunks(j, issue_next):
            wait_gather(j, rows_a, sem_a)
            gather(j + 1, rows_b, sem_b)
            unpack_dst(j, 0)
            sync_scatter(rows_a, 0)
            wait_gather(j + 1, rows_b, sem_b)
            if issue_next:
                gather(j + 2, rows_a, sem_a)
            unpack_dst(j + 1, 1)
            sync_scatter(rows_b, 1)

        def body(i, carry):
            two_chunks(2 * i, True)
            return carry

        lax.fori_loop(0, (_NCHUNK - 2) // 2, body, 0)
        two_chunks(_NCHUNK - 2, False)
        plsc.subcore_barrier()

        @pl.when(c == 0)
        def _():
            pltpu.sync_copy(acc.at[pl.ds(s * _RSTEP, _RCNT)],
                            out0_hbm.at[pl.ds(s * _RSTEP, _RCNT)])

        @pl.when(c == 1)
        def _():
            pltpu.sync_copy(acc.at[pl.ds(s * _RSTEP, _RCNT)],
                            out1_hbm.at[pl.ds(s * _RSTEP, _RCNT)])

    return seg_sum(xn, src_w, dstp_w, zrows)


_R = 2000


def _dot(a, b):
    return lax.dot_general(a, b, (((1,), (0,)), ((), ())),
                           precision=lax.Precision.HIGHEST,
                           preferred_element_type=jnp.float32)


def _tc_matmul(x, w):
    def body(x_ref, w_ref, o_ref):
        o_ref[...] = _dot(x_ref[...], w_ref[...])

    return pl.pallas_call(
        body,
        grid=(_N // _R,),
        in_specs=[pl.BlockSpec((_R, _D), lambda i: (i, 0)),
                  pl.BlockSpec((_D, _D), lambda i: (0, 0))],
        out_specs=pl.BlockSpec((_R, _D), lambda i: (i, 0)),
        out_shape=jax.ShapeDtypeStruct((_N, _D), jnp.float32),
    )(x, w)


def _tc_combine(x, agg2, w_self, b, gamma, beta, w_next):
    agg_a, agg_b = agg2
    has_next = w_next is not None

    def body(x_ref, agga_ref, aggb_ref, ws_ref, b_ref, g_ref, be_ref, *rest):
        if has_next:
            wn_ref, h_ref, y_ref = rest
        else:
            (h_ref,) = rest
        t = (_dot(x_ref[...], ws_ref[...])
             + agga_ref[...] + aggb_ref[...] + b_ref[...])
        mu = jnp.mean(t, axis=-1, keepdims=True)
        d = t - mu
        var = jnp.mean(d * d, axis=-1, keepdims=True)
        h = d * lax.rsqrt(var + 1e-5) * g_ref[...] + be_ref[...]
        h = jnp.maximum(h, 0.0)
        h_ref[...] = h
        if has_next:
            y_ref[...] = _dot(h, wn_ref[...])

    row_spec = pl.BlockSpec((_R, _D), lambda i: (i, 0))
    full_spec = pl.BlockSpec((_D, _D), lambda i: (0, 0))
    vec_spec = pl.BlockSpec((1, _D), lambda i: (0, 0))
    in_specs = [row_spec, row_spec, row_spec,
                full_spec, vec_spec, vec_spec, vec_spec]
    args = [x, agg_a, agg_b, w_self, b, gamma, beta]
    out_shape = jax.ShapeDtypeStruct((_N, _D), jnp.float32)
    if has_next:
        in_specs.append(full_spec)
        args.append(w_next)
        return pl.pallas_call(
            body,
            grid=(_N // _R,),
            in_specs=in_specs,
            out_specs=(row_spec, row_spec),
            out_shape=(out_shape, out_shape),
        )(*args)
    return pl.pallas_call(
        body,
        grid=(_N // _R,),
        in_specs=in_specs,
        out_specs=row_spec,
        out_shape=out_shape,
    )(*args)


def kernel(features, edges,
           W_self_0, W_neigh_0, b_0, gamma_0, beta_0,
           W_self_1, W_neigh_1, b_1, gamma_1, beta_1,
           W_self_2, W_neigh_2, b_2, gamma_2, beta_2):
    i_pad = lax.broadcasted_iota(jnp.int32, (_NW, _NPAD), 1)
    w_pad = lax.broadcasted_iota(jnp.int32, (_NW, _NPAD), 0)
    pad_src = (w_pad * 997 + i_pad * 13) % _N
    pad_dst = _N + (i_pad % _NDUMMY)
    src_w = jnp.concatenate(
        [edges[0].reshape(_NW, _E // _NW), pad_src], axis=1
    ).reshape(_NW, _NCHUNK, _CH)
    dst3 = jnp.concatenate(
        [edges[1].reshape(_NW, _E // _NW), pad_dst], axis=1
    ).reshape(_NW, _NCHUNK, _CH)
    dstp_w = (dst3[:, :, : _CH // 2] | (dst3[:, :, _CH // 2:] << 16)
              ).reshape(_NW, _NCHUNK * _CH // 2)
    zrows = jnp.zeros((_RCNT, _D), jnp.float32)
    b0, g0, be0 = b_0.reshape(1, _D), gamma_0.reshape(1, _D), beta_0.reshape(1, _D)
    b1, g1, be1 = b_1.reshape(1, _D), gamma_1.reshape(1, _D), beta_1.reshape(1, _D)
    b2, g2, be2 = b_2.reshape(1, _D), gamma_2.reshape(1, _D), beta_2.reshape(1, _D)

    xn0 = _tc_matmul(features, W_neigh_0)
    agg0 = _sc_segment_sum(xn0, src_w, dstp_w, zrows)
    h1, xn1 = _tc_combine(features, agg0, W_self_0, b0, g0, be0, W_neigh_1)
    agg1 = _sc_segment_sum(xn1, src_w, dstp_w, zrows)
    h2, xn2 = _tc_combine(h1, agg1, W_self_1, b1, g1, be1, W_neigh_2)
    agg2 = _sc_segment_sum(xn2, src_w, dstp_w, zrows)
    return _tc_combine(h2, agg2, W_self_2, b2, g2, be2, None)
